# baseline (speedup 1.0000x reference)
import jax
import jax.numpy as jnp
from jax import lax
from jax.experimental import pallas as pl
from jax.experimental.pallas import tpu as pltpu


def _gather_block_kernel(idx_ref, off_ref, tab_ref, out_ref, *, fields, pairs_per_field):
    bsub = idx_ref.shape[0]
    d = out_ref.shape[1] // fields
    g = idx_ref[...] + off_ref[...]
    bases = lax.broadcasted_iota(jnp.int32, (1, fields), 1) * pairs_per_field
    lp_bf_all = (lax.shift_right_logical(g, 1) - bases).astype(jnp.bfloat16)
    odd_all = (g & 1) == 1
    pair_iota = lax.broadcasted_iota(
        jnp.int32, (bsub, pairs_per_field), 1
    ).astype(jnp.bfloat16)
    for f in range(fields):
        base = f * pairs_per_field
        lp_bf = lp_bf_all[:, f : f + 1]
        onehot = jnp.maximum(1.0 - jnp.abs(lp_bf - pair_iota), 0.0)
        sub = tab_ref[base : base + pairs_per_field, :]
        pair = jnp.dot(onehot, sub, preferred_element_type=jnp.float32)
        odd = odd_all[:, f : f + 1]
        res = jnp.where(odd, pair[:, d:], pair[:, :d])
        out_ref[:, f * d : (f + 1) * d] = res.astype(jnp.bfloat16)


def kernel(x, embedding_weight, offsets):
    B, F = x.shape
    V, D = embedding_weight.shape
    rows_per_field = V // F

    packed = embedding_weight.astype(jnp.bfloat16).reshape(V // 2, 2 * D)
    off_row = offsets.astype(jnp.int32).reshape(1, F)

    BSUB = 8192
    assert B % BSUB == 0

    out = pl.pallas_call(
        lambda i, of, t, o: _gather_block_kernel(
            i, of, t, o, fields=F, pairs_per_field=rows_per_field // 2
        ),
        out_shape=jax.ShapeDtypeStruct((B, F * D), jnp.bfloat16),
        grid=(B // BSUB,),
        in_specs=[
            pl.BlockSpec((BSUB, F), lambda i: (i, 0)),
            pl.BlockSpec((1, F), lambda i: (0, 0)),
            pl.BlockSpec((V // 2, 2 * D), lambda i: (0, 0)),
        ],
        out_specs=pl.BlockSpec((BSUB, F * D), lambda i: (i, 0)),
        compiler_params=pltpu.CompilerParams(
            dimension_semantics=("parallel",),
            vmem_limit_bytes=60 * 1024 * 1024,
        ),
    )(x.astype(jnp.int32), off_row, packed)

    return out.astype(jnp.float32).reshape(B, F, D)


# speedup vs baseline: 1.1673x; 1.1673x over previous
import jax
import jax.numpy as jnp
from jax import lax
from jax.experimental import pallas as pl
from jax.experimental.pallas import tpu as pltpu


def _gather_block_kernel(idx_ref, off_ref, tab_ref, out_ref, *, fields, pairs_per_field):
    bsub = idx_ref.shape[0]
    d = out_ref.shape[1] // fields
    g = idx_ref[...] + off_ref[...]
    bases = lax.broadcasted_iota(jnp.int32, (1, fields), 1) * pairs_per_field
    lp_bf_all = (lax.shift_right_logical(g, 1) - bases).astype(jnp.bfloat16)
    odd_all = (g & 1) == 1
    pair_iota = lax.broadcasted_iota(
        jnp.int32, (bsub, pairs_per_field), 1
    ).astype(jnp.bfloat16)
    for j in range(fields // 2):
        reses = []
        for f in (2 * j, 2 * j + 1):
            base = f * pairs_per_field
            lp_bf = lp_bf_all[:, f : f + 1]
            onehot = jnp.maximum(1.0 - jnp.abs(lp_bf - pair_iota), 0.0)
            sub = tab_ref[base : base + pairs_per_field, :]
            pair = jnp.dot(onehot, sub, preferred_element_type=jnp.float32)
            odd = odd_all[:, f : f + 1]
            res = jnp.where(odd, pair[:, d:], pair[:, :d])
            reses.append(res.astype(jnp.bfloat16))
        out_ref[:, 2 * j * d : (2 * j + 2) * d] = jnp.concatenate(reses, axis=1)


def kernel(x, embedding_weight, offsets):
    B, F = x.shape
    V, D = embedding_weight.shape
    rows_per_field = V // F

    packed = embedding_weight.astype(jnp.bfloat16).reshape(V // 2, 2 * D)
    off_row = offsets.astype(jnp.int32).reshape(1, F)

    BSUB = 4096
    assert B % BSUB == 0

    out = pl.pallas_call(
        lambda i, of, t, o: _gather_block_kernel(
            i, of, t, o, fields=F, pairs_per_field=rows_per_field // 2
        ),
        out_shape=jax.ShapeDtypeStruct((B, F * D), jnp.bfloat16),
        grid=(B // BSUB,),
        in_specs=[
            pl.BlockSpec((BSUB, F), lambda i: (i, 0)),
            pl.BlockSpec((1, F), lambda i: (0, 0)),
            pl.BlockSpec((V // 2, 2 * D), lambda i: (0, 0)),
        ],
        out_specs=pl.BlockSpec((BSUB, F * D), lambda i: (i, 0)),
        compiler_params=pltpu.CompilerParams(
            dimension_semantics=("parallel",),
            vmem_limit_bytes=60 * 1024 * 1024,
        ),
    )(x.astype(jnp.int32), off_row, packed)

    return out.astype(jnp.float32).reshape(B, F, D)
